# Initial kernel scaffold; baseline (speedup 1.0000x reference)
import jax
import jax.numpy as jnp
from jax.experimental import pallas as pl


def kernel(logits, targets):
    raise NotImplementedError("write your pallas kernel here")


# speedup vs baseline: 1.3218x; 1.3218x over previous
import jax
import jax.numpy as jnp
import numpy as np
from jax.experimental import pallas as pl

_THRESHOLD = 0.95
_ALPHA = 0.1
_NUM_CLASSES = 1000
_BATCH = 16384

_MISS = np.float32(_ALPHA / _NUM_CLASSES)
_HIT = np.float32(np.float32(1.0 - _ALPHA) + _MISS)

_BLOCK_ROWS = 256


def _body(x_ref, t_ref, out_ref, mask_ref):
    x = x_ref[...]
    m = jnp.max(x, axis=1, keepdims=True)
    e = jnp.exp(x - m)
    s = jnp.sum(e, axis=1, keepdims=True)
    conf = 1.0 / s
    msk = conf > _THRESHOLD
    idx = jax.lax.broadcasted_iota(jnp.int32, x.shape, 1)
    pred = jnp.min(jnp.where(x == m, idx, _NUM_CLASSES), axis=1, keepdims=True)
    label = jnp.where(msk, pred, t_ref[...])
    out_ref[...] = jnp.where(idx == label, _HIT, _MISS)
    mask_ref[...] = msk.astype(jnp.float32)


def kernel(logits, targets):
    b, c = logits.shape
    r = _BLOCK_ROWS
    grid = (b // r,)
    tgt2d = targets.astype(jnp.int32).reshape(b, 1)
    smooth, mask2d = pl.pallas_call(
        _body,
        grid=grid,
        in_specs=[
            pl.BlockSpec((r, c), lambda i: (i, 0)),
            pl.BlockSpec((r, 1), lambda i: (i, 0)),
        ],
        out_specs=[
            pl.BlockSpec((r, c), lambda i: (i, 0)),
            pl.BlockSpec((r, 1), lambda i: (i, 0)),
        ],
        out_shape=[
            jax.ShapeDtypeStruct((b, c), jnp.float32),
            jax.ShapeDtypeStruct((b, 1), jnp.float32),
        ],
    )(logits, tgt2d)
    return smooth, mask2d.reshape(b)


# speedup vs baseline: 1.4860x; 1.1243x over previous
import jax
import jax.numpy as jnp
import numpy as np
from jax.experimental import pallas as pl

_THRESHOLD = 0.95
_ALPHA = 0.1
_NUM_CLASSES = 1000
_BATCH = 16384

_MISS = np.float32(_ALPHA / _NUM_CLASSES)
_HIT = np.float32(np.float32(1.0 - _ALPHA) + _MISS)

_BLOCK_ROWS = 512


def _body(x_ref, t_ref, out_ref, mask_ref):
    x = x_ref[...]
    m = jnp.max(x, axis=1, keepdims=True)
    e = jnp.exp(x - m)
    s = jnp.sum(e, axis=1, keepdims=True)
    conf = 1.0 / s
    msk = conf > _THRESHOLD
    idx = jax.lax.broadcasted_iota(jnp.int32, x.shape, 1)
    pred = jnp.min(jnp.where(x == m, idx, _NUM_CLASSES), axis=1, keepdims=True)
    label = jnp.where(msk, pred, t_ref[...])
    out_ref[...] = jnp.where(idx == label, _HIT, _MISS)
    mask_ref[...] = msk.astype(jnp.float32)


def kernel(logits, targets):
    b, c = logits.shape
    r = _BLOCK_ROWS
    grid = (b // r,)
    tgt2d = targets.astype(jnp.int32).reshape(b, 1)
    smooth, mask2d = pl.pallas_call(
        _body,
        grid=grid,
        in_specs=[
            pl.BlockSpec((r, c), lambda i: (i, 0)),
            pl.BlockSpec((r, 1), lambda i: (i, 0)),
        ],
        out_specs=[
            pl.BlockSpec((r, c), lambda i: (i, 0)),
            pl.BlockSpec((r, 1), lambda i: (i, 0)),
        ],
        out_shape=[
            jax.ShapeDtypeStruct((b, c), jnp.float32),
            jax.ShapeDtypeStruct((b, 1), jnp.float32),
        ],
    )(logits, tgt2d)
    return smooth, mask2d.reshape(b)


# speedup vs baseline: 1.5656x; 1.0535x over previous
import jax
import jax.numpy as jnp
import numpy as np
from jax.experimental import pallas as pl

_THRESHOLD = 0.95
_ALPHA = 0.1
_NUM_CLASSES = 1000
_BATCH = 16384

_MISS = np.float32(_ALPHA / _NUM_CLASSES)
_HIT = np.float32(np.float32(1.0 - _ALPHA) + _MISS)

_BLOCK_ROWS = 1024


def _body(x_ref, t_ref, out_ref, mask_ref):
    x = x_ref[...]
    m = jnp.max(x, axis=1, keepdims=True)
    e = jnp.exp(x - m)
    s = jnp.sum(e, axis=1, keepdims=True)
    conf = 1.0 / s
    msk = conf > _THRESHOLD
    idx = jax.lax.broadcasted_iota(jnp.int32, x.shape, 1)
    pred = jnp.min(jnp.where(x == m, idx, _NUM_CLASSES), axis=1, keepdims=True)
    label = jnp.where(msk, pred, t_ref[...])
    out_ref[...] = jnp.where(idx == label, _HIT, _MISS)
    mask_ref[...] = msk.astype(jnp.float32)


def kernel(logits, targets):
    b, c = logits.shape
    r = _BLOCK_ROWS
    grid = (b // r,)
    tgt2d = targets.astype(jnp.int32).reshape(b, 1)
    smooth, mask2d = pl.pallas_call(
        _body,
        grid=grid,
        in_specs=[
            pl.BlockSpec((r, c), lambda i: (i, 0)),
            pl.BlockSpec((r, 1), lambda i: (i, 0)),
        ],
        out_specs=[
            pl.BlockSpec((r, c), lambda i: (i, 0)),
            pl.BlockSpec((r, 1), lambda i: (i, 0)),
        ],
        out_shape=[
            jax.ShapeDtypeStruct((b, c), jnp.float32),
            jax.ShapeDtypeStruct((b, 1), jnp.float32),
        ],
    )(logits, tgt2d)
    return smooth, mask2d.reshape(b)


# speedup vs baseline: 1.5697x; 1.0027x over previous
import jax
import jax.numpy as jnp
import numpy as np
from jax.experimental import pallas as pl

_THRESHOLD = 0.95
_ALPHA = 0.1
_NUM_CLASSES = 1000
_BATCH = 16384

_MISS = np.float32(_ALPHA / _NUM_CLASSES)
_HIT = np.float32(np.float32(1.0 - _ALPHA) + _MISS)

_BLOCK_ROWS = 2048


def _body(x_ref, t_ref, out_ref, mask_ref):
    x = x_ref[...]
    m = jnp.max(x, axis=1, keepdims=True)
    e = jnp.exp(x - m)
    s = jnp.sum(e, axis=1, keepdims=True)
    conf = 1.0 / s
    msk = conf > _THRESHOLD
    idx = jax.lax.broadcasted_iota(jnp.int32, x.shape, 1)
    pred = jnp.min(jnp.where(x == m, idx, _NUM_CLASSES), axis=1, keepdims=True)
    label = jnp.where(msk, pred, t_ref[...])
    out_ref[...] = jnp.where(idx == label, _HIT, _MISS)
    mask_ref[...] = msk.astype(jnp.float32)


def kernel(logits, targets):
    b, c = logits.shape
    r = _BLOCK_ROWS
    grid = (b // r,)
    tgt2d = targets.astype(jnp.int32).reshape(b, 1)
    smooth, mask2d = pl.pallas_call(
        _body,
        grid=grid,
        in_specs=[
            pl.BlockSpec((r, c), lambda i: (i, 0)),
            pl.BlockSpec((r, 1), lambda i: (i, 0)),
        ],
        out_specs=[
            pl.BlockSpec((r, c), lambda i: (i, 0)),
            pl.BlockSpec((r, 1), lambda i: (i, 0)),
        ],
        out_shape=[
            jax.ShapeDtypeStruct((b, c), jnp.float32),
            jax.ShapeDtypeStruct((b, 1), jnp.float32),
        ],
    )(logits, tgt2d)
    return smooth, mask2d.reshape(b)
